# baseline (speedup 1.0000x reference)
import functools

import jax
import jax.numpy as jnp
from jax.experimental import pallas as pl
from jax.experimental.pallas import tpu as pltpu


def _round_up(x, m):
    return (x + m - 1) // m * m


def _fused_gcn_kernel(x_ref, w_ref, adj_ref, b_ref, o_ref, s_ref):
    j = pl.program_id(1)

    @pl.when(j == 0)
    def _():
        s_ref[...] = jnp.dot(
            x_ref[...], w_ref[...], preferred_element_type=jnp.float32
        ).astype(jnp.bfloat16)

    a = adj_ref[...].astype(jnp.bfloat16)
    acc = jnp.dot(a, s_ref[...], preferred_element_type=jnp.float32)
    o_ref[...] = acc + b_ref[...]


def _fused_gcn_kernel_nobias(x_ref, w_ref, adj_ref, o_ref, s_ref):
    j = pl.program_id(1)

    @pl.when(j == 0)
    def _():
        s_ref[...] = jnp.dot(
            x_ref[...], w_ref[...], preferred_element_type=jnp.float32
        ).astype(jnp.bfloat16)

    a = adj_ref[...].astype(jnp.bfloat16)
    o_ref[...] = jnp.dot(a, s_ref[...], preferred_element_type=jnp.float32)


def kernel(x, weight, adj, bias=None):
    n, f_in = x.shape
    f_out = weight.shape[1]
    f32 = jnp.float32

    f_out_p = _round_up(f_out, 128)
    f_in_p = _round_up(f_in, 128)
    n_p = _round_up(n, 128)

    x_p = x.astype(f32)
    if (n, f_in) != (n_p, f_in_p):
        x_p = jnp.zeros((n_p, f_in_p), f32).at[:n, :f_in].set(x_p)
    w_p = weight.astype(f32)
    if (f_in, f_out) != (f_in_p, f_out_p):
        w_p = jnp.zeros((f_in_p, f_out_p), f32).at[:f_in, :f_out].set(w_p)
    adj_p = adj
    if n != n_p:
        adj_p = jnp.zeros((n_p, n_p), adj.dtype).at[:n, :n].set(adj)
    has_bias = bias is not None
    if has_bias:
        b_p = bias.reshape(1, f_out).astype(f32)
        if f_out != f_out_p:
            b_p = jnp.zeros((1, f_out_p), f32).at[:, :f_out].set(b_p)

    tm = max(d for d in (512, 256, 128) if n_p % d == 0)
    n_tiles = n_p // tm
    n_par = 2 if n_tiles % 2 == 0 else 1
    half = n_tiles // n_par

    ws = (n_p * f_in_p * 4
          + f_in_p * f_out_p * 4
          + 2 * tm * n_p * adj_p.dtype.itemsize
          + n_p * f_out_p * 2
          + 2 * tm * f_out_p * 4
          + f_out_p * 4)

    in_specs = [
        pl.BlockSpec((n_p, f_in_p), lambda i, j: (0, 0)),
        pl.BlockSpec((f_in_p, f_out_p), lambda i, j: (0, 0)),
        pl.BlockSpec((tm, n_p), lambda i, j: (i * half + j, 0)),
    ]
    if has_bias:
        in_specs.append(pl.BlockSpec((1, f_out_p), lambda i, j: (0, 0)))
        kfn = _fused_gcn_kernel
        args = (x_p, w_p, adj_p, b_p)
    else:
        kfn = _fused_gcn_kernel_nobias
        args = (x_p, w_p, adj_p)

    out = pl.pallas_call(
        kfn,
        out_shape=jax.ShapeDtypeStruct((n_p, f_out_p), f32),
        grid=(n_par, half),
        in_specs=in_specs,
        out_specs=pl.BlockSpec((tm, f_out_p), lambda i, j: (i * half + j, 0)),
        scratch_shapes=[pltpu.VMEM((n_p, f_out_p), jnp.bfloat16)],
        compiler_params=pltpu.CompilerParams(
            dimension_semantics=("parallel", "arbitrary"),
            vmem_limit_bytes=int(min(max(int(ws * 1.25), 16 << 20), 56 << 20))),
        cost_estimate=pl.CostEstimate(
            flops=2 * n_p * n_p * f_out_p + 2 * n_par * n_p * f_in_p * f_out_p,
            transcendentals=0,
            bytes_accessed=int(n_p * n_p * adj_p.dtype.itemsize
                               + n_par * n_p * f_in_p * 4
                               + n_p * f_out_p * 4)),
    )(*args)

    if (n, f_out) != (n_p, f_out_p):
        out = out[:n, :f_out]
    return out
